# Initial kernel scaffold; baseline (speedup 1.0000x reference)
import jax
import jax.numpy as jnp
from jax.experimental import pallas as pl


def kernel(atom_type_index, atom_code_index, residue_code_index, residue_sequence_index, atom_type_table, atom_code_table, residue_code_table, residue_index_table):
    raise NotImplementedError("write your pallas kernel here")


# speedup vs baseline: 1.7218x; 1.7218x over previous
import functools

import jax
import jax.numpy as jnp
from jax import lax
from jax.experimental import pallas as pl
from jax.experimental.pallas import tpu as pltpu
from jax.experimental.pallas import tpu_sc as plsc

N = 100000
D = 32
NW = 32
G = 112
CHUNKS = 28
B_PER_W = G * CHUNKS
N_PAD = NW * B_PER_W
N_ROWS = N_PAD // G


def _sc_embed(i0, i1, i2, i3, t0, t1, t2, t3):
    mesh = plsc.VectorSubcoreMesh(core_axis_name="c", subcore_axis_name="s")

    @functools.partial(
        pl.kernel,
        mesh=mesh,
        compiler_params=pltpu.CompilerParams(use_tc_tiling_on_sc=False),
        out_type=jax.ShapeDtypeStruct((N_PAD, 4 * D), jnp.float32),
        scratch_types=[
            pltpu.VMEM((4, CHUNKS, G), jnp.int32),
            pltpu.VMEM((4, G, D), jnp.float32),
            pltpu.SemaphoreType.DMA,
            pltpu.SemaphoreType.DMA,
        ],
    )
    def k(i0h, i1h, i2h, i3h, t0h, t1h, t2h, t3h, out, idx_v, rows_v, gsem, ssem):
        wid = lax.axis_index("s") * 2 + lax.axis_index("c")
        rb = wid * CHUNKS
        ab = wid * B_PER_W

        pltpu.sync_copy(i0h.at[pl.ds(rb, CHUNKS)], idx_v.at[0])
        pltpu.sync_copy(i1h.at[pl.ds(rb, CHUNKS)], idx_v.at[1])
        pltpu.sync_copy(i2h.at[pl.ds(rb, CHUNKS)], idx_v.at[2])
        pltpu.sync_copy(i3h.at[pl.ds(rb, CHUNKS)], idx_v.at[3])

        def body(g, carry):
            c0 = pltpu.async_copy(t0h.at[idx_v.at[0, g]], rows_v.at[0], gsem)
            c1 = pltpu.async_copy(t1h.at[idx_v.at[1, g]], rows_v.at[1], gsem)
            c2 = pltpu.async_copy(t2h.at[idx_v.at[2, g]], rows_v.at[2], gsem)
            c3 = pltpu.async_copy(t3h.at[idx_v.at[3, g]], rows_v.at[3], gsem)
            c0.wait(); c1.wait(); c2.wait(); c3.wait()
            base = ab + g * G
            s0 = pltpu.async_copy(rows_v.at[0], out.at[pl.ds(base, G), pl.ds(0 * D, D)], ssem)
            s1 = pltpu.async_copy(rows_v.at[1], out.at[pl.ds(base, G), pl.ds(1 * D, D)], ssem)
            s2 = pltpu.async_copy(rows_v.at[2], out.at[pl.ds(base, G), pl.ds(2 * D, D)], ssem)
            s3 = pltpu.async_copy(rows_v.at[3], out.at[pl.ds(base, G), pl.ds(3 * D, D)], ssem)
            s0.wait(); s1.wait(); s2.wait(); s3.wait()
            return carry

        lax.fori_loop(0, CHUNKS, body, 0)

    return k(i0, i1, i2, i3, t0, t1, t2, t3)


def kernel(atom_type_index, atom_code_index, residue_code_index, residue_sequence_index,
           atom_type_table, atom_code_table, residue_code_table, residue_index_table):
    pad = N_PAD - N
    i0 = jnp.pad(atom_type_index, (0, pad)).reshape(N_ROWS, G)
    i1 = jnp.pad(atom_code_index, (0, pad)).reshape(N_ROWS, G)
    i2 = jnp.pad(residue_code_index, (0, pad)).reshape(N_ROWS, G)
    i3 = jnp.pad(residue_sequence_index, (0, pad)).reshape(N_ROWS, G)
    out = _sc_embed(i0, i1, i2, i3, atom_type_table, atom_code_table,
                    residue_code_table, residue_index_table)
    return out[:N]
